# baseline (speedup 1.0000x reference)
import jax
import jax.numpy as jnp
from jax import lax
from jax.experimental import pallas as pl
from jax.experimental.pallas import tpu as pltpu
from jax.experimental.pallas import tpu_sc as plsc

D_MODEL = 64
SEQ = 200
NC, NS = 2, 16
NW = NC * NS
LANES = 16
BW = 128
DH, DL = 8, 8
GL = BW // LANES
VPR = D_MODEL // LANES
TSTR = D_MODEL + 1
SCH = 2
NCHK = SEQ // SCH


def _emb_body(x_hbm, table_hbm, pos_hbm, out_hbm,
              idx_v, pos_v, gbuf, tbuf, obuf,
              isem0, isem1, gsem0, gsem1, osem0, osem1):
    wid = lax.axis_index("s") * NC + lax.axis_index("c")

    pltpu.sync_copy(pos_hbm, pos_v)

    isems = (isem0, isem1)
    gsems = (gsem0, gsem1)
    osems = (osem0, osem1)

    def idx_copy(c, buf):
        return pltpu.make_async_copy(
            x_hbm.at[pl.ds(c * SCH, SCH), wid], idx_v.at[buf], isems[buf])

    def gather_copy(sl, buf):
        return pltpu.make_async_copy(
            table_hbm.at[idx_v.at[buf, sl]],
            gbuf.at[buf, pl.ds(sl * BW, BW)], gsems[buf])

    def out_copy(c, sl, buf):
        return pltpu.make_async_copy(
            obuf.at[buf, sl], out_hbm.at[c * SCH + sl, slice(None), wid],
            osems[buf])

    idx_copy(0, 0).start()
    idx_copy(1, 1).start()
    for b in range(2):
        idx_copy(b, b).wait()
        for sl in range(SCH):
            gather_copy(sl, b).start()

    iota = lax.iota(jnp.int32, LANES)

    def chunk(t, b):
        c = 2 * t + b
        for sl in range(SCH):
            gather_copy(sl, b).wait()
        @pl.when(t > 0)
        def _():
            for sl in range(SCH):
                out_copy(c - 2, sl, b).wait()
        @pl.when(c + 2 < NCHK)
        def _():
            idx_copy(c + 2, b).start()

        bvec = jnp.full((LANES,), b, jnp.int32)
        for sl in range(SCH):
            s = c * SCH + sl
            svec = jnp.full((LANES,), s, jnp.int32)

            @plsc.parallel_loop(0, DH, step=1, unroll=2)
            def _(dh):
                for dl in range(DL):
                    d = dh * DL + dl
                    dvec = jnp.full((LANES,), d, jnp.int32)
                    ps = plsc.load_gather(pos_v, [svec, dvec])
                    for g in range(GL):
                        rows = iota + (sl * BW + g * LANES)
                        rv = plsc.load_gather(gbuf, [bvec, rows, dvec])
                        obuf[b, sl, dh, dl, pl.ds(g * LANES, LANES)] = rv + ps

        @pl.when(c + 2 < NCHK)
        def _():
            idx_copy(c + 2, b).wait()
            for sl in range(SCH):
                gather_copy(sl, b).start()

        for sl in range(SCH):
            out_copy(c, sl, b).start()

    def step(t, _):
        chunk(t, 0)
        chunk(t, 1)
        return 0

    lax.fori_loop(0, NCHK // 2, step, 0)

    for b in range(2):
        for sl in range(SCH):
            out_copy(NCHK - 2 + b, sl, b).wait()


def kernel(x, token_table, pos_table):
    B, S = x.shape
    xt = x.astype(jnp.int32).T.reshape(S, NW, BW)

    mesh = plsc.VectorSubcoreMesh(core_axis_name="c", subcore_axis_name="s")
    out5 = pl.kernel(
        _emb_body,
        out_type=jax.ShapeDtypeStruct((SEQ, DH, NW, DL, BW), jnp.float32),
        mesh=mesh,
        compiler_params=pltpu.CompilerParams(
            use_tc_tiling_on_sc=False, needs_layout_passes=False,
            disable_bounds_checks=True),
        scratch_types=[
            pltpu.VMEM((2, SCH, BW), jnp.int32),
            pltpu.VMEM((SEQ, D_MODEL), jnp.float32),
            pltpu.VMEM((2, SCH * BW, D_MODEL), jnp.float32),
            pltpu.VMEM((BW, TSTR), jnp.float32),
            pltpu.VMEM((2, SCH, DH, DL, BW), jnp.float32),
            pltpu.SemaphoreType.DMA,
            pltpu.SemaphoreType.DMA,
            pltpu.SemaphoreType.DMA,
            pltpu.SemaphoreType.DMA,
            pltpu.SemaphoreType.DMA,
            pltpu.SemaphoreType.DMA,
        ],
    )(xt, token_table, pos_table)

    return out5.transpose((2, 4, 0, 1, 3)).reshape(B, S, D_MODEL)


# speedup vs baseline: 1.6862x; 1.6862x over previous
import jax
import jax.numpy as jnp
from jax import lax
from jax.experimental import pallas as pl
from jax.experimental.pallas import tpu as pltpu
from jax.experimental.pallas import tpu_sc as plsc

D_MODEL = 64
SEQ = 200
NC, NS = 2, 16
NW = NC * NS
LANES = 16
BW = 128
DH, DL = 8, 8
GL = BW // LANES
VPR = D_MODEL // LANES
TSTR = D_MODEL + 1
SCH = 2
NCHK = SEQ // SCH


def _emb_body(x_hbm, table_hbm, pos_hbm, out_hbm,
              idx_v, pos_v, gbuf, tbuf, obuf,
              isem0, isem1, gsem0, gsem1, osem0, osem1):
    wid = lax.axis_index("s") * NC + lax.axis_index("c")

    pltpu.sync_copy(pos_hbm, pos_v)

    isems = (isem0, isem1)
    gsems = (gsem0, gsem1)
    osems = (osem0, osem1)

    def idx_copy(c, buf):
        return pltpu.make_async_copy(
            x_hbm.at[pl.ds(c * SCH, SCH), wid], idx_v.at[buf], isems[buf])

    def gather_copy(sl, buf):
        return pltpu.make_async_copy(
            table_hbm.at[idx_v.at[buf, sl]],
            gbuf.at[buf, pl.ds(sl * BW, BW)], gsems[buf])

    def out_copy(c, sl, buf):
        return pltpu.make_async_copy(
            obuf.at[buf, sl], out_hbm.at[c * SCH + sl, slice(None), wid],
            osems[buf])

    idx_copy(0, 0).start()
    idx_copy(1, 1).start()
    for b in range(2):
        idx_copy(b, b).wait()
        for sl in range(SCH):
            gather_copy(sl, b).start()

    iota = lax.iota(jnp.int32, LANES)

    def chunk(t, b):
        c = 2 * t + b
        for sl in range(SCH):
            gather_copy(sl, b).wait()
        @pl.when(t > 0)
        def _():
            for sl in range(SCH):
                out_copy(c - 2, sl, b).wait()
        @pl.when(c + 2 < NCHK)
        def _():
            idx_copy(c + 2, b).start()

        for sl in range(SCH):
            s = c * SCH + sl

            pv = [pos_v[s, pl.ds(j * LANES, LANES)] for j in range(VPR)]

            @plsc.parallel_loop(0, BW, step=1, unroll=2)
            def _(rr):
                for j in range(VPR):
                    sl16 = pl.ds(j * LANES, LANES)
                    tbuf[rr, sl16] = gbuf[b, sl * BW + rr, sl16] + pv[j]

            @plsc.parallel_loop(0, DH, step=1, unroll=1)
            def _(dh):
                for dl in range(DL):
                    d = dh * DL + dl
                    dvec = jnp.full((LANES,), d, jnp.int32)
                    for g in range(GL):
                        rows = iota + g * LANES
                        rv = plsc.load_gather(tbuf, [rows, dvec])
                        obuf[b, sl, dh, dl, pl.ds(g * LANES, LANES)] = rv

        @pl.when(c + 2 < NCHK)
        def _():
            idx_copy(c + 2, b).wait()
            for sl in range(SCH):
                gather_copy(sl, b).start()

        for sl in range(SCH):
            out_copy(c, sl, b).start()

    def step(t, _):
        chunk(t, 0)
        chunk(t, 1)
        return 0

    lax.fori_loop(0, NCHK // 2, step, 0)

    for b in range(2):
        for sl in range(SCH):
            out_copy(NCHK - 2 + b, sl, b).wait()


def kernel(x, token_table, pos_table):
    B, S = x.shape
    xt = x.astype(jnp.int32).T.reshape(S, NW, BW)

    mesh = plsc.VectorSubcoreMesh(core_axis_name="c", subcore_axis_name="s")
    out5 = pl.kernel(
        _emb_body,
        out_type=jax.ShapeDtypeStruct((SEQ, DH, NW, DL, BW), jnp.float32),
        mesh=mesh,
        compiler_params=pltpu.CompilerParams(
            use_tc_tiling_on_sc=False, needs_layout_passes=False,
            disable_bounds_checks=True),
        scratch_types=[
            pltpu.VMEM((2, SCH, BW), jnp.int32),
            pltpu.VMEM((SEQ, D_MODEL), jnp.float32),
            pltpu.VMEM((2, SCH * BW, D_MODEL), jnp.float32),
            pltpu.VMEM((BW, TSTR), jnp.float32),
            pltpu.VMEM((2, SCH, DH, DL, BW), jnp.float32),
            pltpu.SemaphoreType.DMA,
            pltpu.SemaphoreType.DMA,
            pltpu.SemaphoreType.DMA,
            pltpu.SemaphoreType.DMA,
            pltpu.SemaphoreType.DMA,
            pltpu.SemaphoreType.DMA,
        ],
    )(xt, token_table, pos_table)

    return out5.transpose((2, 4, 0, 1, 3)).reshape(B, S, D_MODEL)
